# baseline (speedup 1.0000x reference)
import functools

import jax
import jax.numpy as jnp
from jax import lax
from jax.experimental import pallas as pl
from jax.experimental.pallas import tpu as pltpu
from jax.experimental.pallas import tpu_sc as plsc

D = 128
NUM_F = 16
NC = 2
NS = 16
NW = NC * NS
TPAD = 2048

_VOCABS = [256, 256, 256, 35, 370, 9, 2, 2, 21, 14, 7, 275, 57, 2, 295, 69]
_OFFSETS = [0]
for _v in _VOCABS[:-1]:
    _OFFSETS.append(_OFFSETS[-1] + _v)

ROWS_PER_TASK = 128
RING = 5
G_AHEAD = 3
W_LAG = 2
NCOPY = 2
ADD_UNROLL = 8


def _sc_embed(xf, tcat, offs):
    BF = xf.shape[0]
    per_w = BF // NW
    n_tasks = per_w // ROWS_PER_TASK
    stripe = TPAD // NS
    mesh = plsc.VectorSubcoreMesh(core_axis_name="c", subcore_axis_name="s")

    @functools.partial(
        pl.kernel,
        mesh=mesh,
        out_type=jax.ShapeDtypeStruct((BF, D), jnp.float32),
        scratch_types=[
            pltpu.VMEM((per_w,), jnp.int32),
            pltpu.VMEM((16,), jnp.int32),
            pltpu.VMEM((RING, ROWS_PER_TASK, D), jnp.float32),
            pltpu.VMEM_SHARED((NCOPY * TPAD, D), jnp.float32),
            pltpu.SemaphoreType.DMA,
            pltpu.SemaphoreType.DMA,
        ],
    )
    def k(xf_hbm, t_hbm, off_hbm, out_hbm,
          idx_v, off_v, rows_v, tab_s, sem_g, sem_w):
        cid = lax.axis_index("c")
        sid = lax.axis_index("s")
        wid = sid * NC + cid
        base = wid * per_w

        for r in range(NCOPY):
            pltpu.sync_copy(t_hbm.at[pl.ds(sid * stripe, stripe)],
                            tab_s.at[pl.ds(r * TPAD + sid * stripe, stripe)])
        pltpu.sync_copy(xf_hbm.at[pl.ds(base, per_w)], idx_v)
        pltpu.sync_copy(off_hbm, off_v)
        offv = off_v[...] + lax.rem(sid, NCOPY) * TPAD

        def add_body(j, carry):
            for u in range(ADD_UNROLL):
                o = (j * ADD_UNROLL + u) * NUM_F
                idx_v[pl.ds(o, NUM_F)] = idx_v[pl.ds(o, NUM_F)] + offv
            return carry

        lax.fori_loop(0, per_w // (NUM_F * ADD_UNROLL), add_body, 0)
        plsc.subcore_barrier()

        def gather(t):
            slot = lax.rem(t, RING)
            pltpu.async_copy(
                tab_s.at[idx_v.at[pl.ds(t * ROWS_PER_TASK, ROWS_PER_TASK)]],
                rows_v.at[slot], sem_g)

        def drain_write():
            pltpu.make_async_copy(
                out_hbm.at[pl.ds(base, ROWS_PER_TASK)], rows_v.at[0],
                sem_w).wait()

        for kk in range(G_AHEAD):
            gather(kk)

        def step(t, carry):
            @pl.when(t >= W_LAG)
            def _():
                drain_write()

            @pl.when(t + G_AHEAD < n_tasks)
            def _():
                gather(t + G_AHEAD)

            slot = lax.rem(t, RING)
            pltpu.make_async_copy(
                out_hbm.at[pl.ds(base, ROWS_PER_TASK)], rows_v.at[slot],
                sem_g).wait()
            pltpu.async_copy(
                rows_v.at[slot],
                out_hbm.at[pl.ds(base + t * ROWS_PER_TASK, ROWS_PER_TASK)],
                sem_w)
            return carry

        lax.fori_loop(0, n_tasks, step, 0)

        for _ in range(W_LAG):
            drain_write()

    return k(xf, tcat, offs)


def kernel(x, table_0, table_1, table_2, table_3, table_4, table_5,
           table_6, table_7, table_8, table_9, table_10, table_11,
           table_12, table_13, table_14, table_15):
    tables = (table_0, table_1, table_2, table_3, table_4, table_5,
              table_6, table_7, table_8, table_9, table_10, table_11,
              table_12, table_13, table_14, table_15)
    B = x.shape[0]
    xf = x.astype(jnp.int32).reshape(B * NUM_F)
    tcat = jnp.concatenate(tables, axis=0)
    pad = TPAD - tcat.shape[0]
    tcat = jnp.pad(tcat, ((0, pad), (0, 0)))
    offs = jnp.asarray(_OFFSETS, dtype=jnp.int32)
    out_rows = _sc_embed(xf, tcat, offs)
    return out_rows.reshape(B, NUM_F * D)


# speedup vs baseline: 1.0050x; 1.0050x over previous
import functools

import jax
import jax.numpy as jnp
from jax import lax
from jax.experimental import pallas as pl
from jax.experimental.pallas import tpu as pltpu
from jax.experimental.pallas import tpu_sc as plsc

D = 128
NUM_F = 16
NC = 2
NS = 16
NW = NC * NS
TPAD = 2048

_VOCABS = [256, 256, 256, 35, 370, 9, 2, 2, 21, 14, 7, 275, 57, 2, 295, 69]
_OFFSETS = [0]
for _v in _VOCABS[:-1]:
    _OFFSETS.append(_OFFSETS[-1] + _v)

ROWS_PER_TASK = 128
RING = 6
G_AHEAD = 3
W_LAG = 3
ADD_UNROLL = 8


def _sc_embed(xf, tcat, offs):
    BF = xf.shape[0]
    per_w = BF // NW
    n_tasks = per_w // ROWS_PER_TASK
    stripe = TPAD // NS
    mesh = plsc.VectorSubcoreMesh(core_axis_name="c", subcore_axis_name="s")

    @functools.partial(
        pl.kernel,
        mesh=mesh,
        out_type=jax.ShapeDtypeStruct((BF, D), jnp.float32),
        scratch_types=[
            pltpu.VMEM((per_w,), jnp.int32),
            pltpu.VMEM((16,), jnp.int32),
            pltpu.VMEM((RING, ROWS_PER_TASK, D), jnp.float32),
            pltpu.VMEM_SHARED((TPAD, D), jnp.float32),
            pltpu.SemaphoreType.DMA,
            pltpu.SemaphoreType.DMA,
        ],
    )
    def k(xf_hbm, t_hbm, off_hbm, out_hbm,
          idx_v, off_v, rows_v, tab_s, sem_g, sem_w):
        cid = lax.axis_index("c")
        sid = lax.axis_index("s")
        wid = sid * NC + cid
        base = wid * per_w

        h_idx = pltpu.async_copy(xf_hbm.at[pl.ds(base, per_w)], idx_v, sem_g)
        h_tab = pltpu.async_copy(t_hbm.at[pl.ds(sid * stripe, stripe)],
                                 tab_s.at[pl.ds(sid * stripe, stripe)], sem_w)
        pltpu.sync_copy(off_hbm, off_v)
        offv = off_v[...]
        h_idx.wait()

        def add_body(j, carry):
            for u in range(ADD_UNROLL):
                o = (j * ADD_UNROLL + u) * NUM_F
                idx_v[pl.ds(o, NUM_F)] = idx_v[pl.ds(o, NUM_F)] + offv
            return carry

        lax.fori_loop(0, per_w // (NUM_F * ADD_UNROLL), add_body, 0)
        h_tab.wait()
        plsc.subcore_barrier()

        def gather(t):
            slot = lax.rem(t, RING)
            pltpu.async_copy(
                tab_s.at[idx_v.at[pl.ds(t * ROWS_PER_TASK, ROWS_PER_TASK)]],
                rows_v.at[slot], sem_g)

        def drain_write():
            pltpu.make_async_copy(
                out_hbm.at[pl.ds(base, ROWS_PER_TASK)], rows_v.at[0],
                sem_w).wait()

        for kk in range(G_AHEAD):
            gather(kk)

        def step(t, carry):
            @pl.when(t >= W_LAG)
            def _():
                drain_write()

            @pl.when(t + G_AHEAD < n_tasks)
            def _():
                gather(t + G_AHEAD)

            slot = lax.rem(t, RING)
            pltpu.make_async_copy(
                out_hbm.at[pl.ds(base, ROWS_PER_TASK)], rows_v.at[slot],
                sem_g).wait()
            pltpu.async_copy(
                rows_v.at[slot],
                out_hbm.at[pl.ds(base + t * ROWS_PER_TASK, ROWS_PER_TASK)],
                sem_w)
            return carry

        lax.fori_loop(0, n_tasks, step, 0)

        for _ in range(W_LAG):
            drain_write()

    return k(xf, tcat, offs)


def kernel(x, table_0, table_1, table_2, table_3, table_4, table_5,
           table_6, table_7, table_8, table_9, table_10, table_11,
           table_12, table_13, table_14, table_15):
    tables = (table_0, table_1, table_2, table_3, table_4, table_5,
              table_6, table_7, table_8, table_9, table_10, table_11,
              table_12, table_13, table_14, table_15)
    B = x.shape[0]
    xf = x.astype(jnp.int32).reshape(B * NUM_F)
    tcat = jnp.concatenate(tables, axis=0)
    pad = TPAD - tcat.shape[0]
    tcat = jnp.pad(tcat, ((0, pad), (0, 0)))
    offs = jnp.asarray(_OFFSETS, dtype=jnp.int32)
    out_rows = _sc_embed(xf, tcat, offs)
    return out_rows.reshape(B, NUM_F * D)
